# baseline (speedup 1.0000x reference)
import jax
import jax.numpy as jnp
from jax import lax
from jax.experimental import pallas as pl
from jax.experimental.pallas import tpu as pltpu
from jax.experimental.pallas import tpu_sc as plsc

_N = 1000000
_D = 64
_NW = 32
_ROWS = 31248
_TAIL = _N - _NW * _ROWS


def _copy_body(states_hbm, out_hbm):
    c = lax.axis_index("c")
    s = lax.axis_index("s")
    wid = s * 2 + c
    base = wid * _ROWS
    pltpu.sync_copy(states_hbm.at[pl.ds(base, _ROWS)],
                    out_hbm.at[pl.ds(base, _ROWS)])

    @pl.when(wid == 0)
    def _():
        pltpu.sync_copy(states_hbm.at[pl.ds(_NW * _ROWS, _TAIL)],
                        out_hbm.at[pl.ds(_NW * _ROWS, _TAIL)])


def kernel(states, idx, updated):
    mesh = plsc.VectorSubcoreMesh(core_axis_name="c", subcore_axis_name="s")
    copied = pl.kernel(
        _copy_body,
        out_type=jax.ShapeDtypeStruct((_N, _D), jnp.float32),
        mesh=mesh,
    )(states)
    return copied.at[idx].set(updated)


# speedup vs baseline: 6.6765x; 6.6765x over previous
import jax
import jax.numpy as jnp
from jax import lax
from jax.experimental import pallas as pl
from jax.experimental.pallas import tpu as pltpu
from jax.experimental.pallas import tpu_sc as plsc

_N = 1000000
_D = 64
_NW = 32
_ROWS = 31248
_TAIL = _N - _NW * _ROWS


_CHUNK = 504
_NCHUNK = _ROWS // _CHUNK


def _copy_body(states_hbm, out_hbm, buf, sem_in, sem_out):
    c = lax.axis_index("c")
    s = lax.axis_index("s")
    wid = s * 2 + c
    base = wid * _ROWS

    def in_copy(i, slot):
        return pltpu.make_async_copy(
            states_hbm.at[pl.ds(base + i * _CHUNK, _CHUNK)],
            buf.at[slot], sem_in.at[slot])

    def out_copy(i, slot):
        return pltpu.make_async_copy(
            buf.at[slot],
            out_hbm.at[pl.ds(base + i * _CHUNK, _CHUNK)],
            sem_out.at[slot])

    in_copy(0, 0).start()

    def step(i, _):
        slot = lax.rem(i, 2)
        nxt = lax.rem(i + 1, 2)

        @pl.when(i + 1 < _NCHUNK)
        def _():
            @pl.when(i + 1 >= 2)
            def _():
                out_copy(i - 1, nxt).wait()
            in_copy(i + 1, nxt).start()

        in_copy(i, slot).wait()
        out_copy(i, slot).start()
        return 0

    lax.fori_loop(0, _NCHUNK, step, 0)
    out_copy(_NCHUNK - 2, 0 if _NCHUNK % 2 == 0 else 1).wait()
    out_copy(_NCHUNK - 1, 1 if _NCHUNK % 2 == 0 else 0).wait()

    @pl.when(wid == 0)
    def _():
        pltpu.sync_copy(states_hbm.at[pl.ds(_NW * _ROWS, _TAIL)],
                        out_hbm.at[pl.ds(_NW * _ROWS, _TAIL)])


def kernel(states, idx, updated):
    mesh = plsc.VectorSubcoreMesh(core_axis_name="c", subcore_axis_name="s")
    copied = pl.kernel(
        _copy_body,
        out_type=jax.ShapeDtypeStruct((_N, _D), jnp.float32),
        mesh=mesh,
        scratch_types=[
            pltpu.VMEM((2, _CHUNK, _D), jnp.float32),
            pltpu.SemaphoreType.DMA((2,)),
            pltpu.SemaphoreType.DMA((2,)),
        ],
    )(states)
    return copied.at[idx].set(updated)


# speedup vs baseline: 12.4523x; 1.8651x over previous
import jax
import jax.numpy as jnp
from jax import lax
from jax.experimental import pallas as pl
from jax.experimental.pallas import tpu as pltpu
from jax.experimental.pallas import tpu_sc as plsc

_N = 1000000
_D = 64
_B = 16384
_NW = 32
_ROWS = 31248
_TAIL = _N - _NW * _ROWS
_RANGE_LAST = _ROWS + _TAIL
_CAP = 2048
_CCH = 504
_NCH = _ROWS // _CCH


def _body(states_hbm, idx_hbm, upd_hbm, out_hbm,
          idx_v, tbuf, pbuf, tfin, pfin, posmap, cbuf, rows,
          sem_in, sem_out, sem_rg, sem_rs):
    c = lax.axis_index("c")
    s = lax.axis_index("s")
    wid = s * 2 + c
    base = wid * _ROWS
    myrange = jnp.where(wid == _NW - 1, _RANGE_LAST, _ROWS)
    lanes = lax.broadcasted_iota(jnp.int32, (16,), 0)

    pltpu.sync_copy(idx_hbm, idx_v)

    def scan_step(k, cnt):
        t = idx_v[pl.ds(k * 16, 16)] - base
        m = (t >= 0) & (t < myrange)
        pos = lanes + k * 16
        off = plsc.cumsum(jnp.where(m, 1, 0).astype(jnp.int32))
        dest = jnp.maximum(cnt + off - 1, 0)
        plsc.store_scatter(tbuf, [dest], t, mask=m)
        plsc.store_scatter(pbuf, [dest], pos, mask=m)
        return jnp.minimum(cnt + jnp.max(off), _CAP - 16)

    cnt = lax.fori_loop(0, _B // 16, scan_step, jnp.int32(0))
    nch = (cnt + 15) // 16

    def dedup_step(j, _):
        valid = (j * 16 + lanes) < cnt
        tt = jnp.where(valid, tbuf[pl.ds(j * 16, 16)], 0)
        pp = jnp.where(valid, pbuf[pl.ds(j * 16, 16)], -1)

        def cond(lost):
            return jnp.any(lost)

        def body(lost):
            plsc.store_scatter(posmap, [tt], pp, mask=lost)
            g = plsc.load_gather(posmap, [tt], mask=valid)
            return valid & (g < pp)

        lax.while_loop(cond, body, valid)
        return 0

    lax.fori_loop(0, nch, dedup_step, 0)

    def win_step(j, cnt2):
        valid = (j * 16 + lanes) < cnt
        tt = jnp.where(valid, tbuf[pl.ds(j * 16, 16)], 0)
        pp = jnp.where(valid, pbuf[pl.ds(j * 16, 16)], -1)
        g = plsc.load_gather(posmap, [tt], mask=valid)
        keep = valid & (g == pp)
        off = plsc.cumsum(jnp.where(keep, 1, 0).astype(jnp.int32))
        dest = jnp.maximum(cnt2 + off - 1, 0)
        plsc.store_scatter(tfin, [dest], tt + base, mask=keep)
        plsc.store_scatter(pfin, [dest], pp, mask=keep)
        return jnp.minimum(cnt2 + jnp.max(off), _CAP - 16)

    cnt2 = lax.fori_loop(0, nch, win_step, jnp.int32(0))
    nj = (cnt2 + 15) // 16

    @pl.when((cnt2 > 0) & (cnt2 % 16 != 0))
    def _():
        fill = jnp.full((16,), cnt2 - 1, jnp.int32)
        tfin[pl.ds(cnt2, 16)] = plsc.load_gather(tfin, [fill])
        pfin[pl.ds(cnt2, 16)] = plsc.load_gather(pfin, [fill])

    def in_copy(i, slot):
        return pltpu.make_async_copy(
            states_hbm.at[pl.ds(base + i * _CCH, _CCH)],
            cbuf.at[slot], sem_in.at[slot])

    def out_copy(i, slot):
        return pltpu.make_async_copy(
            cbuf.at[slot],
            out_hbm.at[pl.ds(base + i * _CCH, _CCH)], sem_out.at[slot])

    in_copy(0, 0).start()

    def copy_step(i, _):
        slot = lax.rem(i, 2)
        nxt = lax.rem(i + 1, 2)

        @pl.when(i + 1 < _NCH)
        def _():
            @pl.when(i >= 1)
            def _():
                out_copy(i - 1, nxt).wait()
            in_copy(i + 1, nxt).start()

        in_copy(i, slot).wait()
        out_copy(i, slot).start()
        return 0

    lax.fori_loop(0, _NCH, copy_step, 0)
    out_copy(_NCH - 2, _NCH % 2).wait()
    out_copy(_NCH - 1, 1 - _NCH % 2).wait()

    @pl.when(wid == _NW - 1)
    def _():
        pltpu.sync_copy(states_hbm.at[pl.ds(_NW * _ROWS, _TAIL)],
                        out_hbm.at[pl.ds(_NW * _ROWS, _TAIL)])

    def row_gather(j, slot):
        pp16 = pfin[pl.ds(j * 16, 16)]
        return pltpu.make_async_copy(upd_hbm.at[pp16], rows.at[slot],
                                     sem_rg.at[slot])

    def row_scatter(j, slot):
        tt16 = tfin[pl.ds(j * 16, 16)]
        return pltpu.make_async_copy(rows.at[slot], out_hbm.at[tt16],
                                     sem_rs.at[slot])

    @pl.when(nj > 0)
    def _():
        row_gather(0, 0).start()

        def p4_step(j, _):
            slot = lax.rem(j, 2)
            nxt = lax.rem(j + 1, 2)

            @pl.when(j + 1 < nj)
            def _():
                @pl.when(j >= 1)
                def _():
                    row_scatter(j - 1, nxt).wait()
                row_gather(j + 1, nxt).start()

            row_gather(j, slot).wait()
            row_scatter(j, slot).start()
            return 0

        lax.fori_loop(0, nj, p4_step, 0)

        @pl.when(nj >= 2)
        def _():
            row_scatter(nj - 2, lax.rem(nj - 2, 2)).wait()
        row_scatter(nj - 1, lax.rem(nj - 1, 2)).wait()


def kernel(states, idx, updated):
    mesh = plsc.VectorSubcoreMesh(core_axis_name="c", subcore_axis_name="s")
    return pl.kernel(
        _body,
        out_type=jax.ShapeDtypeStruct((_N, _D), jnp.float32),
        mesh=mesh,
        compiler_params=pltpu.CompilerParams(needs_layout_passes=False,
                                             use_tc_tiling_on_sc=False),
        scratch_types=[
            pltpu.VMEM((_B,), jnp.int32),
            pltpu.VMEM((_CAP,), jnp.int32),
            pltpu.VMEM((_CAP,), jnp.int32),
            pltpu.VMEM((_CAP,), jnp.int32),
            pltpu.VMEM((_CAP,), jnp.int32),
            pltpu.VMEM((_RANGE_LAST,), jnp.int32),
            pltpu.VMEM((2, _CCH, _D), jnp.float32),
            pltpu.VMEM((2, 16, _D), jnp.float32),
            pltpu.SemaphoreType.DMA((2,)),
            pltpu.SemaphoreType.DMA((2,)),
            pltpu.SemaphoreType.DMA((2,)),
            pltpu.SemaphoreType.DMA((2,)),
        ],
    )(states, idx, updated)
